# baseline (speedup 1.0000x reference)
import functools

import jax
import jax.numpy as jnp
from jax import lax
from jax.experimental import pallas as pl
from jax.experimental.pallas import tpu as pltpu
from jax.experimental.pallas import tpu_sc as plsc

_B, _N, _M = 8, 2048, 2048
_L = 16
_NW = 32

_SC_NB = 2
_TC_NB = _B - _SC_NB
_WPB = _NW // _SC_NB
_OWN = _N // _WPB
_KB = 8
_IC = _OWN // _KB
_YC = _M // _L

_NT = 4
_NB = _N // _NT


def _splat(val):
    return jnp.full((_L,), val, jnp.int32)


def _sc_body(recT_hbm, dataT_hbm, dx_hbm, dyp_hbm, own_v, part_v, rows_v,
             out_v, col_v):
    wid = lax.axis_index("s") * 2 + lax.axis_index("c")
    b = wid // _WPB
    q = wid % _WPB
    for c in range(3):
        pltpu.sync_copy(
            recT_hbm.at[pl.ds((b * 3 + c) * _N + q * _OWN, _OWN)],
            own_v.at[pl.ds(c * _OWN, _OWN)])
        pltpu.sync_copy(dataT_hbm.at[pl.ds((b * 3 + c) * _M, _M)],
                        part_v.at[pl.ds(c * _M, _M)])

    inf = jnp.full((_L,), jnp.float32(jnp.inf), jnp.float32)

    def init_col(yc, carry):
        col_v[pl.ds(yc * _L, _L)] = inf
        return carry

    lax.fori_loop(0, _YC, init_col, 0)

    def ic_body(ic, carry):
        base = ic * _KB
        xb = []
        for k in range(_KB):
            xb.append([
                plsc.load_gather(own_v, [_splat(c * _OWN + base + k)])
                for c in range(3)
            ])

        def yc_body(yc, rowaccs):
            y0 = part_v[pl.ds(yc * _L, _L)]
            y1 = part_v[pl.ds(_M + yc * _L, _L)]
            y2 = part_v[pl.ds(2 * _M + yc * _L, _L)]
            cacc = col_v[pl.ds(yc * _L, _L)]
            new = []
            for k in range(_KB):
                d0 = xb[k][0] - y0
                d1 = xb[k][1] - y1
                d2 = xb[k][2] - y2
                d = d0 * d0 + d1 * d1 + d2 * d2
                new.append(jnp.minimum(rowaccs[k], d))
                cacc = jnp.minimum(cacc, d)
            col_v[pl.ds(yc * _L, _L)] = cacc
            return tuple(new)

        rowaccs = lax.fori_loop(0, _YC, yc_body, (inf,) * _KB)
        for k in range(_KB):
            rows_v[pl.ds((base + k) * _L, _L)] = rowaccs[k]
        return carry

    lax.fori_loop(0, _IC, ic_body, 0)

    lanes = lax.iota(jnp.int32, _L)

    def red_body(oc, carry):
        flat0 = (oc * _L + lanes) * _L
        m = plsc.load_gather(rows_v, [flat0])
        for k in range(1, _L):
            m = jnp.minimum(m, plsc.load_gather(rows_v, [flat0 + k]))
        out_v[pl.ds(oc * _L, _L)] = m
        return carry

    lax.fori_loop(0, _OWN // _L, red_body, 0)

    pltpu.sync_copy(out_v, dx_hbm.at[pl.ds(b * _N + q * _OWN, _OWN)])
    pltpu.sync_copy(col_v, dyp_hbm.at[pl.ds((b * _WPB + q) * _M, _M)])


def _sc_chamfer(recT, dataT):
    mesh = plsc.VectorSubcoreMesh(core_axis_name="c", subcore_axis_name="s")
    f = functools.partial(
        pl.kernel,
        mesh=mesh,
        compiler_params=pltpu.CompilerParams(needs_layout_passes=False),
        out_type=[
            jax.ShapeDtypeStruct((_SC_NB * _N,), jnp.float32),
            jax.ShapeDtypeStruct((_SC_NB * _WPB * _M,), jnp.float32),
        ],
        scratch_types=[
            pltpu.VMEM((3 * _OWN,), jnp.float32),
            pltpu.VMEM((3 * _M,), jnp.float32),
            pltpu.VMEM((_OWN * _L,), jnp.float32),
            pltpu.VMEM((_OWN,), jnp.float32),
            pltpu.VMEM((_M,), jnp.float32),
        ],
    )(_sc_body)
    return f(recT.reshape(-1), dataT.reshape(-1))


def _tc_body(x_ref, yt_ref, dx_ref, dy_ref):
    n = pl.program_id(1)
    x = x_ref[0]
    yt = yt_ref[0]
    d0 = x[:, 0:1] - yt[0:1, :]
    d1 = x[:, 1:2] - yt[1:2, :]
    d2 = x[:, 2:3] - yt[2:3, :]
    d = d0 * d0 + d1 * d1 + d2 * d2
    dx_ref[0, 0] = jnp.min(d, axis=1)
    colmin = jnp.min(d, axis=0)

    @pl.when(n == 0)
    def _():
        dy_ref[0, 0] = colmin

    @pl.when(n != 0)
    def _():
        dy_ref[0, 0] = jnp.minimum(dy_ref[0, 0], colmin)


def _tc_chamfer(rec, dataT):
    nb = rec.shape[0]
    dist_x, dist_y = pl.pallas_call(
        _tc_body,
        grid=(nb, _NT),
        in_specs=[
            pl.BlockSpec((1, _NB, 3), lambda b, n: (b, n, 0)),
            pl.BlockSpec((1, 3, _M), lambda b, n: (b, 0, 0)),
        ],
        out_specs=[
            pl.BlockSpec((1, 1, _NB), lambda b, n: (b * _NT + n, 0, 0)),
            pl.BlockSpec((1, 1, _M), lambda b, n: (b, 0, 0)),
        ],
        out_shape=[
            jax.ShapeDtypeStruct((nb * _NT, 1, _NB), jnp.float32),
            jax.ShapeDtypeStruct((nb, 1, _M), jnp.float32),
        ],
    )(rec, dataT)
    mean_x = jnp.mean(dist_x.reshape(nb, _N), axis=1)
    mean_y = jnp.mean(dist_y.reshape(nb, _M), axis=1)
    return jnp.maximum(mean_y, mean_x)


def kernel(rec, data):
    recT2 = jnp.transpose(rec[:_SC_NB], (0, 2, 1))
    dataT2 = jnp.transpose(data[:_SC_NB], (0, 2, 1))
    dataT6 = jnp.transpose(data[_SC_NB:], (0, 2, 1))
    dx_flat, dyp_flat = _sc_chamfer(recT2, dataT2)
    tc_pb = _tc_chamfer(rec[_SC_NB:], dataT6)
    sc_pbs = []
    for b in range(_SC_NB):
        mean_x = jnp.mean(dx_flat[b * _N:(b + 1) * _N])
        dyb = dyp_flat[b * _WPB * _M:(b + 1) * _WPB * _M].reshape(_WPB, _M)
        mean_y = jnp.mean(jnp.min(dyb, axis=0))
        sc_pbs.append(jnp.maximum(mean_y, mean_x))
    per_batch = jnp.concatenate([jnp.stack(sc_pbs), tc_pb])
    return jnp.mean(per_batch)


# speedup vs baseline: 1.0747x; 1.0747x over previous
import functools

import jax
import jax.numpy as jnp
from jax import lax
from jax.experimental import pallas as pl
from jax.experimental.pallas import tpu as pltpu
from jax.experimental.pallas import tpu_sc as plsc

_B, _N, _M = 8, 2048, 2048
_L = 16
_NW = 32

_SC_NB = 2
_TC_NB = _B - _SC_NB
_WPB = _NW // _SC_NB
_OWN = _N // _WPB
_KB = 8
_IC = _OWN // _KB
_YC = _M // _L

_NT = 4
_NB = _N // _NT


def _splat(val):
    return jnp.full((_L,), val, jnp.int32)


def _sc_body(recT_hbm, dataT_hbm, dx_hbm, dyp_hbm, own_v, part_v, rows_v,
             out_v, col_v):
    wid = lax.axis_index("s") * 2 + lax.axis_index("c")
    b = wid // _WPB
    q = wid % _WPB
    for c in range(3):
        pltpu.sync_copy(
            recT_hbm.at[pl.ds((b * 3 + c) * _N + q * _OWN, _OWN)],
            own_v.at[pl.ds(c * _OWN, _OWN)])
        pltpu.sync_copy(dataT_hbm.at[pl.ds((b * 3 + c) * _M, _M)],
                        part_v.at[pl.ds(c * _M, _M)])

    inf = jnp.full((_L,), jnp.float32(jnp.inf), jnp.float32)

    def init_col(yc, carry):
        col_v[pl.ds(yc * _L, _L)] = inf
        return carry

    lax.fori_loop(0, _YC, init_col, 0)

    def ic_body(ic, carry):
        base = ic * _KB
        xb = []
        for k in range(_KB):
            xb.append([
                plsc.load_gather(own_v, [_splat(c * _OWN + base + k)])
                for c in range(3)
            ])

        def yc_body(yc, rowaccs):
            y0 = part_v[pl.ds(yc * _L, _L)]
            y1 = part_v[pl.ds(_M + yc * _L, _L)]
            y2 = part_v[pl.ds(2 * _M + yc * _L, _L)]
            cacc = col_v[pl.ds(yc * _L, _L)]
            new = []
            for k in range(_KB):
                d0 = xb[k][0] - y0
                d1 = xb[k][1] - y1
                d2 = xb[k][2] - y2
                d = d0 * d0 + d1 * d1 + d2 * d2
                new.append(jnp.minimum(rowaccs[k], d))
                cacc = jnp.minimum(cacc, d)
            col_v[pl.ds(yc * _L, _L)] = cacc
            return tuple(new)

        rowaccs = lax.fori_loop(0, _YC, yc_body, (inf,) * _KB)
        for k in range(_KB):
            rows_v[pl.ds((base + k) * _L, _L)] = rowaccs[k]
        return carry

    lax.fori_loop(0, _IC, ic_body, 0)

    lanes = lax.iota(jnp.int32, _L)

    def red_body(oc, carry):
        flat0 = (oc * _L + lanes) * _L
        m = plsc.load_gather(rows_v, [flat0])
        for k in range(1, _L):
            m = jnp.minimum(m, plsc.load_gather(rows_v, [flat0 + k]))
        out_v[pl.ds(oc * _L, _L)] = m
        return carry

    lax.fori_loop(0, _OWN // _L, red_body, 0)

    pltpu.sync_copy(out_v, dx_hbm.at[pl.ds(b * _N + q * _OWN, _OWN)])
    pltpu.sync_copy(col_v, dyp_hbm.at[pl.ds((b * _WPB + q) * _M, _M)])


def _sc_chamfer(recT, dataT):
    mesh = plsc.VectorSubcoreMesh(core_axis_name="c", subcore_axis_name="s")
    f = functools.partial(
        pl.kernel,
        mesh=mesh,
        compiler_params=pltpu.CompilerParams(needs_layout_passes=False),
        out_type=[
            jax.ShapeDtypeStruct((_SC_NB * _N,), jnp.float32),
            jax.ShapeDtypeStruct((_SC_NB * _WPB * _M,), jnp.float32),
        ],
        scratch_types=[
            pltpu.VMEM((3 * _OWN,), jnp.float32),
            pltpu.VMEM((3 * _M,), jnp.float32),
            pltpu.VMEM((_OWN * _L,), jnp.float32),
            pltpu.VMEM((_OWN,), jnp.float32),
            pltpu.VMEM((_M,), jnp.float32),
        ],
    )(_sc_body)
    dx, dyp = f(recT.reshape(-1), dataT.reshape(-1))
    dx = dx.reshape(_SC_NB, _N)
    dyp = dyp.reshape(_SC_NB, _WPB, _M)
    return dx, jnp.min(dyp, axis=1)


def _tc_body(x_ref, yt_ref, dx_ref, dy_ref):
    n = pl.program_id(1)
    x = x_ref[0]
    yt = yt_ref[0]
    d0 = x[:, 0:1] - yt[0:1, :]
    d1 = x[:, 1:2] - yt[1:2, :]
    d2 = x[:, 2:3] - yt[2:3, :]
    d = d0 * d0 + d1 * d1 + d2 * d2
    dx_ref[0, 0] = jnp.min(d, axis=1)
    colmin = jnp.min(d, axis=0)

    @pl.when(n == 0)
    def _():
        dy_ref[0, 0] = colmin

    @pl.when(n != 0)
    def _():
        dy_ref[0, 0] = jnp.minimum(dy_ref[0, 0], colmin)


def _tc_chamfer(rec, dataT):
    nb = rec.shape[0]
    dist_x, dist_y = pl.pallas_call(
        _tc_body,
        grid=(nb, _NT),
        in_specs=[
            pl.BlockSpec((1, _NB, 3), lambda b, n: (b, n, 0)),
            pl.BlockSpec((1, 3, _M), lambda b, n: (b, 0, 0)),
        ],
        out_specs=[
            pl.BlockSpec((1, 1, _NB), lambda b, n: (b * _NT + n, 0, 0)),
            pl.BlockSpec((1, 1, _M), lambda b, n: (b, 0, 0)),
        ],
        out_shape=[
            jax.ShapeDtypeStruct((nb * _NT, 1, _NB), jnp.float32),
            jax.ShapeDtypeStruct((nb, 1, _M), jnp.float32),
        ],
    )(rec, dataT)
    return dist_x.reshape(nb, _N), dist_y.reshape(nb, _M)


def kernel(rec, data):
    recT2 = jnp.transpose(rec[:_SC_NB], (0, 2, 1))
    dataT2 = jnp.transpose(data[:_SC_NB], (0, 2, 1))
    dataT6 = jnp.transpose(data[_SC_NB:], (0, 2, 1))
    sc_dx, sc_dy = _sc_chamfer(recT2, dataT2)
    tc_dx, tc_dy = _tc_chamfer(rec[_SC_NB:], dataT6)
    dist_x = jnp.concatenate([sc_dx, tc_dx], axis=0)
    dist_y = jnp.concatenate([sc_dy, tc_dy], axis=0)
    per_batch = jnp.maximum(jnp.mean(dist_y, axis=1), jnp.mean(dist_x, axis=1))
    return jnp.mean(per_batch)


# speedup vs baseline: 1.0893x; 1.0135x over previous
import functools

import jax
import jax.numpy as jnp
from jax import lax
from jax.experimental import pallas as pl
from jax.experimental.pallas import tpu as pltpu
from jax.experimental.pallas import tpu_sc as plsc

_B, _N, _M = 8, 2048, 2048
_L = 16
_NW = 32

_SC_NB = 2
_TC_NB = _B - _SC_NB
_WPB = _NW // _SC_NB
_OWN = _N // _WPB
_KB = 8
_IC = _OWN // _KB
_YC = _M // _L

_NT = 2
_NB = _N // _NT


def _splat(val):
    return jnp.full((_L,), val, jnp.int32)


def _sc_body(recT_hbm, dataT_hbm, dx_hbm, dyp_hbm, own_v, part_v, rows_v,
             out_v, col_v):
    wid = lax.axis_index("s") * 2 + lax.axis_index("c")
    b = wid // _WPB
    q = wid % _WPB
    for c in range(3):
        pltpu.sync_copy(
            recT_hbm.at[pl.ds((b * 3 + c) * _N + q * _OWN, _OWN)],
            own_v.at[pl.ds(c * _OWN, _OWN)])
        pltpu.sync_copy(dataT_hbm.at[pl.ds((b * 3 + c) * _M, _M)],
                        part_v.at[pl.ds(c * _M, _M)])

    inf = jnp.full((_L,), jnp.float32(jnp.inf), jnp.float32)

    def init_col(yc, carry):
        col_v[pl.ds(yc * _L, _L)] = inf
        return carry

    lax.fori_loop(0, _YC, init_col, 0)

    def ic_body(ic, carry):
        base = ic * _KB
        xb = []
        for k in range(_KB):
            xb.append([
                plsc.load_gather(own_v, [_splat(c * _OWN + base + k)])
                for c in range(3)
            ])

        def yc_body(yc, rowaccs):
            y0 = part_v[pl.ds(yc * _L, _L)]
            y1 = part_v[pl.ds(_M + yc * _L, _L)]
            y2 = part_v[pl.ds(2 * _M + yc * _L, _L)]
            cacc = col_v[pl.ds(yc * _L, _L)]
            new = []
            for k in range(_KB):
                d0 = xb[k][0] - y0
                d1 = xb[k][1] - y1
                d2 = xb[k][2] - y2
                d = d0 * d0 + d1 * d1 + d2 * d2
                new.append(jnp.minimum(rowaccs[k], d))
                cacc = jnp.minimum(cacc, d)
            col_v[pl.ds(yc * _L, _L)] = cacc
            return tuple(new)

        rowaccs = lax.fori_loop(0, _YC, yc_body, (inf,) * _KB)
        for k in range(_KB):
            rows_v[pl.ds((base + k) * _L, _L)] = rowaccs[k]
        return carry

    lax.fori_loop(0, _IC, ic_body, 0)

    lanes = lax.iota(jnp.int32, _L)

    def red_body(oc, carry):
        flat0 = (oc * _L + lanes) * _L
        m = plsc.load_gather(rows_v, [flat0])
        for k in range(1, _L):
            m = jnp.minimum(m, plsc.load_gather(rows_v, [flat0 + k]))
        out_v[pl.ds(oc * _L, _L)] = m
        return carry

    lax.fori_loop(0, _OWN // _L, red_body, 0)

    pltpu.sync_copy(out_v, dx_hbm.at[pl.ds(b * _N + q * _OWN, _OWN)])
    pltpu.sync_copy(col_v, dyp_hbm.at[pl.ds((b * _WPB + q) * _M, _M)])


def _sc_chamfer(recT, dataT):
    mesh = plsc.VectorSubcoreMesh(core_axis_name="c", subcore_axis_name="s")
    f = functools.partial(
        pl.kernel,
        mesh=mesh,
        compiler_params=pltpu.CompilerParams(needs_layout_passes=False),
        out_type=[
            jax.ShapeDtypeStruct((_SC_NB * _N,), jnp.float32),
            jax.ShapeDtypeStruct((_SC_NB * _WPB * _M,), jnp.float32),
        ],
        scratch_types=[
            pltpu.VMEM((3 * _OWN,), jnp.float32),
            pltpu.VMEM((3 * _M,), jnp.float32),
            pltpu.VMEM((_OWN * _L,), jnp.float32),
            pltpu.VMEM((_OWN,), jnp.float32),
            pltpu.VMEM((_M,), jnp.float32),
        ],
    )(_sc_body)
    dx, dyp = f(recT.reshape(-1), dataT.reshape(-1))
    dx = dx.reshape(_SC_NB, _N)
    dyp = dyp.reshape(_SC_NB, _WPB, _M)
    return dx, jnp.min(dyp, axis=1)


def _tc_body(x_ref, yt_ref, dx_ref, dy_ref):
    n = pl.program_id(1)
    x = x_ref[0]
    yt = yt_ref[0]
    d0 = x[:, 0:1] - yt[0:1, :]
    d1 = x[:, 1:2] - yt[1:2, :]
    d2 = x[:, 2:3] - yt[2:3, :]
    d = d0 * d0 + d1 * d1 + d2 * d2
    dx_ref[0, 0] = jnp.min(d, axis=1)
    colmin = jnp.min(d, axis=0)

    @pl.when(n == 0)
    def _():
        dy_ref[0, 0] = colmin

    @pl.when(n != 0)
    def _():
        dy_ref[0, 0] = jnp.minimum(dy_ref[0, 0], colmin)


def _tc_chamfer(rec, dataT):
    nb = rec.shape[0]
    dist_x, dist_y = pl.pallas_call(
        _tc_body,
        grid=(nb, _NT),
        in_specs=[
            pl.BlockSpec((1, _NB, 3), lambda b, n: (b, n, 0)),
            pl.BlockSpec((1, 3, _M), lambda b, n: (b, 0, 0)),
        ],
        out_specs=[
            pl.BlockSpec((1, 1, _NB), lambda b, n: (b * _NT + n, 0, 0)),
            pl.BlockSpec((1, 1, _M), lambda b, n: (b, 0, 0)),
        ],
        out_shape=[
            jax.ShapeDtypeStruct((nb * _NT, 1, _NB), jnp.float32),
            jax.ShapeDtypeStruct((nb, 1, _M), jnp.float32),
        ],
    )(rec, dataT)
    return dist_x.reshape(nb, _N), dist_y.reshape(nb, _M)


def kernel(rec, data):
    recT2 = jnp.transpose(rec[:_SC_NB], (0, 2, 1))
    dataT2 = jnp.transpose(data[:_SC_NB], (0, 2, 1))
    dataT6 = jnp.transpose(data[_SC_NB:], (0, 2, 1))
    sc_dx, sc_dy = _sc_chamfer(recT2, dataT2)
    tc_dx, tc_dy = _tc_chamfer(rec[_SC_NB:], dataT6)
    dist_x = jnp.concatenate([sc_dx, tc_dx], axis=0)
    dist_y = jnp.concatenate([sc_dy, tc_dy], axis=0)
    per_batch = jnp.maximum(jnp.mean(dist_y, axis=1), jnp.mean(dist_x, axis=1))
    return jnp.mean(per_batch)


# speedup vs baseline: 1.0977x; 1.0077x over previous
import functools

import jax
import jax.numpy as jnp
from jax import lax
from jax.experimental import pallas as pl
from jax.experimental.pallas import tpu as pltpu
from jax.experimental.pallas import tpu_sc as plsc

_B, _N, _M = 8, 2048, 2048
_L = 16
_NW = 32

_SC_NB = 2
_TC_NB = _B - _SC_NB
_WPB = _NW // _SC_NB
_OWN = _N // _WPB
_KB = 8
_IC = _OWN // _KB
_YC = _M // _L

_NT = 1
_NB = _N // _NT


def _splat(val):
    return jnp.full((_L,), val, jnp.int32)


def _sc_body(recT_hbm, dataT_hbm, dx_hbm, dyp_hbm, own_v, part_v, rows_v,
             out_v, col_v):
    wid = lax.axis_index("s") * 2 + lax.axis_index("c")
    b = wid // _WPB
    q = wid % _WPB
    for c in range(3):
        pltpu.sync_copy(
            recT_hbm.at[pl.ds((b * 3 + c) * _N + q * _OWN, _OWN)],
            own_v.at[pl.ds(c * _OWN, _OWN)])
        pltpu.sync_copy(dataT_hbm.at[pl.ds((b * 3 + c) * _M, _M)],
                        part_v.at[pl.ds(c * _M, _M)])

    inf = jnp.full((_L,), jnp.float32(jnp.inf), jnp.float32)

    def init_col(yc, carry):
        col_v[pl.ds(yc * _L, _L)] = inf
        return carry

    lax.fori_loop(0, _YC, init_col, 0)

    def ic_body(ic, carry):
        base = ic * _KB
        xb = []
        for k in range(_KB):
            xb.append([
                plsc.load_gather(own_v, [_splat(c * _OWN + base + k)])
                for c in range(3)
            ])

        def yc_body(yc, rowaccs):
            y0 = part_v[pl.ds(yc * _L, _L)]
            y1 = part_v[pl.ds(_M + yc * _L, _L)]
            y2 = part_v[pl.ds(2 * _M + yc * _L, _L)]
            cacc = col_v[pl.ds(yc * _L, _L)]
            new = []
            for k in range(_KB):
                d0 = xb[k][0] - y0
                d1 = xb[k][1] - y1
                d2 = xb[k][2] - y2
                d = d0 * d0 + d1 * d1 + d2 * d2
                new.append(jnp.minimum(rowaccs[k], d))
                cacc = jnp.minimum(cacc, d)
            col_v[pl.ds(yc * _L, _L)] = cacc
            return tuple(new)

        rowaccs = lax.fori_loop(0, _YC, yc_body, (inf,) * _KB)
        for k in range(_KB):
            rows_v[pl.ds((base + k) * _L, _L)] = rowaccs[k]
        return carry

    lax.fori_loop(0, _IC, ic_body, 0)

    lanes = lax.iota(jnp.int32, _L)

    def red_body(oc, carry):
        flat0 = (oc * _L + lanes) * _L
        m = plsc.load_gather(rows_v, [flat0])
        for k in range(1, _L):
            m = jnp.minimum(m, plsc.load_gather(rows_v, [flat0 + k]))
        out_v[pl.ds(oc * _L, _L)] = m
        return carry

    lax.fori_loop(0, _OWN // _L, red_body, 0)

    pltpu.sync_copy(out_v, dx_hbm.at[pl.ds(b * _N + q * _OWN, _OWN)])
    pltpu.sync_copy(col_v, dyp_hbm.at[pl.ds((b * _WPB + q) * _M, _M)])


def _sc_chamfer(recT, dataT):
    mesh = plsc.VectorSubcoreMesh(core_axis_name="c", subcore_axis_name="s")
    f = functools.partial(
        pl.kernel,
        mesh=mesh,
        compiler_params=pltpu.CompilerParams(needs_layout_passes=False),
        out_type=[
            jax.ShapeDtypeStruct((_SC_NB * _N,), jnp.float32),
            jax.ShapeDtypeStruct((_SC_NB * _WPB * _M,), jnp.float32),
        ],
        scratch_types=[
            pltpu.VMEM((3 * _OWN,), jnp.float32),
            pltpu.VMEM((3 * _M,), jnp.float32),
            pltpu.VMEM((_OWN * _L,), jnp.float32),
            pltpu.VMEM((_OWN,), jnp.float32),
            pltpu.VMEM((_M,), jnp.float32),
        ],
    )(_sc_body)
    dx, dyp = f(recT.reshape(-1), dataT.reshape(-1))
    dx = dx.reshape(_SC_NB, _N)
    dyp = dyp.reshape(_SC_NB, _WPB, _M)
    return dx, jnp.min(dyp, axis=1)


def _tc_body(x_ref, yt_ref, dx_ref, dy_ref):
    n = pl.program_id(1)
    x = x_ref[0]
    yt = yt_ref[0]
    d0 = x[:, 0:1] - yt[0:1, :]
    d1 = x[:, 1:2] - yt[1:2, :]
    d2 = x[:, 2:3] - yt[2:3, :]
    d = d0 * d0 + d1 * d1 + d2 * d2
    dx_ref[0, 0] = jnp.min(d, axis=1)
    colmin = jnp.min(d, axis=0)

    @pl.when(n == 0)
    def _():
        dy_ref[0, 0] = colmin

    @pl.when(n != 0)
    def _():
        dy_ref[0, 0] = jnp.minimum(dy_ref[0, 0], colmin)


def _tc_chamfer(rec, dataT):
    nb = rec.shape[0]
    dist_x, dist_y = pl.pallas_call(
        _tc_body,
        grid=(nb, _NT),
        in_specs=[
            pl.BlockSpec((1, _NB, 3), lambda b, n: (b, n, 0)),
            pl.BlockSpec((1, 3, _M), lambda b, n: (b, 0, 0)),
        ],
        out_specs=[
            pl.BlockSpec((1, 1, _NB), lambda b, n: (b * _NT + n, 0, 0)),
            pl.BlockSpec((1, 1, _M), lambda b, n: (b, 0, 0)),
        ],
        out_shape=[
            jax.ShapeDtypeStruct((nb * _NT, 1, _NB), jnp.float32),
            jax.ShapeDtypeStruct((nb, 1, _M), jnp.float32),
        ],
    )(rec, dataT)
    return dist_x.reshape(nb, _N), dist_y.reshape(nb, _M)


def kernel(rec, data):
    recT2 = jnp.transpose(rec[:_SC_NB], (0, 2, 1))
    dataT2 = jnp.transpose(data[:_SC_NB], (0, 2, 1))
    dataT6 = jnp.transpose(data[_SC_NB:], (0, 2, 1))
    sc_dx, sc_dy = _sc_chamfer(recT2, dataT2)
    tc_dx, tc_dy = _tc_chamfer(rec[_SC_NB:], dataT6)
    dist_x = jnp.concatenate([sc_dx, tc_dx], axis=0)
    dist_y = jnp.concatenate([sc_dy, tc_dy], axis=0)
    per_batch = jnp.maximum(jnp.mean(dist_y, axis=1), jnp.mean(dist_x, axis=1))
    return jnp.mean(per_batch)
